# Initial kernel scaffold; baseline (speedup 1.0000x reference)
import functools

import jax
import jax.numpy as jnp
from jax import lax
from jax.experimental import pallas as pl
from jax.experimental.pallas import tpu as pltpu
from jax.experimental.pallas import tpu_sc as plsc

_B, _L, _D = 1024, 200, 128
_NC, _NS = 2, 16
_NW = _NC * _NS
_ROWS_PER_W = _B // _NW
_NCH = (_L + 15) // 16
_LP = _NCH * 16
_TAIL = _L - (_NCH - 1) * 16

_mesh = plsc.VectorSubcoreMesh(core_axis_name="c", subcore_axis_name="s")


@functools.partial(
    pl.kernel,
    out_type=jax.ShapeDtypeStruct((_B, _L, _D), jnp.float32),
    mesh=_mesh,
    scratch_types=[
        pltpu.VMEM((_LP,), jnp.int32),
        pltpu.VMEM((_LP, _D), jnp.float32),
        pltpu.VMEM((_LP, _D), jnp.float32),
        pltpu.SemaphoreType.DMA,
    ],
)
def _embed_kernel(ids_hbm, char_hbm, pos_hbm, out_hbm, idx_v, cbuf, pbuf, sem):
    wid = lax.axis_index("s") * _NC + lax.axis_index("c")
    row0 = wid * _ROWS_PER_W
    lanes = lax.iota(jnp.int32, 16)

    def row_body(i, unused):
        row = row0 + i
        pltpu.sync_copy(ids_hbm.at[row], idx_v.at[pl.ds(0, _L)])

        carry = jnp.int32(0)
        copies = []
        for j in range(_NCH):
            v = idx_v[pl.ds(j * 16, 16)]
            if j == _NCH - 1:
                v = jnp.where(lanes < _TAIL, v, 0)
            m = (v != 0).astype(jnp.int32)
            pos = (plsc.cumsum(m) + carry) * m
            carry = carry + jnp.sum(m)
            dst = pl.ds(j * 16, 16)
            copies.append(pltpu.async_copy(char_hbm.at[v], cbuf.at[dst], sem))
            copies.append(pltpu.async_copy(pos_hbm.at[pos], pbuf.at[dst], sem))
        for c in copies:
            c.wait()

        def add_body(r, acc):
            for k in range(_D // 16):
                sl = pl.ds(k * 16, 16)
                cbuf[r, sl] = cbuf[r, sl] + pbuf[r, sl]
            return acc

        lax.fori_loop(0, _L, add_body, 0)
        pltpu.sync_copy(cbuf.at[pl.ds(0, _L)], out_hbm.at[row])
        return unused

    lax.fori_loop(0, _ROWS_PER_W, row_body, 0)


def kernel(input_ids, char_table, pos_table):
    return _embed_kernel(input_ids, char_table, pos_table)


# speedup vs baseline: 2.4150x; 2.4150x over previous
import functools

import jax
import jax.numpy as jnp
from jax import lax
from jax.experimental import pallas as pl
from jax.experimental.pallas import tpu as pltpu
from jax.experimental.pallas import tpu_sc as plsc

_B, _L, _D = 1024, 200, 128
_NC, _NS = 2, 16
_NW = _NC * _NS
_ROWS_PER_W = _B // _NW
_NCH = (_L + 15) // 16
_LP = _NCH * 16
_TAIL = _L - (_NCH - 1) * 16

_mesh = plsc.VectorSubcoreMesh(core_axis_name="c", subcore_axis_name="s")


@functools.partial(
    pl.kernel,
    out_type=jax.ShapeDtypeStruct((_B, _L, _D), jnp.float32),
    mesh=_mesh,
    scratch_types=[
        pltpu.VMEM((_LP,), jnp.int32),
        pltpu.VMEM((_LP, _D), jnp.float32),
        pltpu.VMEM((_LP, _D), jnp.float32),
        pltpu.SemaphoreType.DMA,
    ],
    compiler_params=pltpu.CompilerParams(needs_layout_passes=False),
)
def _embed_kernel(ids_hbm, char_hbm, pos_hbm, out_hbm, idx_v, cbuf, pbuf, sem):
    wid = lax.axis_index("s") * _NC + lax.axis_index("c")
    row0 = wid * _ROWS_PER_W
    lanes = lax.iota(jnp.int32, 16)

    def row_body(i, unused):
        row = row0 + i
        pltpu.sync_copy(ids_hbm.at[pl.ds(row * _L, _L)], idx_v.at[pl.ds(0, _L)])

        ones = jnp.full((16,), 1, jnp.int32)
        zeros = jnp.full((16,), 0, jnp.int32)
        carry = zeros
        copies = []
        for j in range(_NCH):
            v = idx_v[pl.ds(j * 16, 16)]
            if j == _NCH - 1:
                v = jnp.where(lanes < _TAIL, v, zeros)
            m = jnp.where(v != 0, ones, zeros)
            pos = (plsc.cumsum(m) + carry) * m
            carry = carry + lax.broadcast_in_dim(jnp.sum(m), (16,), ())
            dst = pl.ds(j * 16, 16)
            copies.append(pltpu.async_copy(char_hbm.at[v], cbuf.at[dst], sem))
            copies.append(pltpu.async_copy(pos_hbm.at[pos], pbuf.at[dst], sem))
        for c in copies:
            c.wait()

        def add_body(r, acc):
            for k in range(_D // 16):
                sl = pl.ds(k * 16, 16)
                cbuf[r, sl] = cbuf[r, sl] + pbuf[r, sl]
            return acc

        lax.fori_loop(0, _L, add_body, 0)
        pltpu.sync_copy(cbuf.at[pl.ds(0, _L)], out_hbm.at[row])
        return unused

    lax.fori_loop(0, _ROWS_PER_W, row_body, 0)


def kernel(input_ids, char_table, pos_table):
    return _embed_kernel(input_ids.reshape(-1), char_table, pos_table)


# speedup vs baseline: 2.6775x; 1.1087x over previous
import functools

import jax
import jax.numpy as jnp
from jax import lax
from jax.experimental import pallas as pl
from jax.experimental.pallas import tpu as pltpu
from jax.experimental.pallas import tpu_sc as plsc

_B, _L, _D = 1024, 200, 128
_MAX_SEQ = 256
_NC, _NS = 2, 16
_NW = _NC * _NS
_ROWS_PER_W = _B // _NW
_NCH = (_L + 15) // 16
_LP = _NCH * 16
_TAIL = _L - (_NCH - 1) * 16

_mesh = plsc.VectorSubcoreMesh(core_axis_name="c", subcore_axis_name="s")


@functools.partial(
    pl.kernel,
    out_type=jax.ShapeDtypeStruct((_B, _L, _D), jnp.float32),
    mesh=_mesh,
    scratch_types=[
        pltpu.VMEM((_LP,), jnp.int32),
        pltpu.VMEM((_LP,), jnp.int32),
        pltpu.VMEM((_LP, _D), jnp.float32),
        pltpu.VMEM((_MAX_SEQ, _D), jnp.float32),
        pltpu.SemaphoreType.DMA,
    ],
    compiler_params=pltpu.CompilerParams(needs_layout_passes=False),
)
def _embed_kernel(ids_hbm, char_hbm, pos_hbm, out_hbm,
                  idx_v, pidx_v, cbuf, ptab_v, sem):
    wid = lax.axis_index("s") * _NC + lax.axis_index("c")
    row0 = wid * _ROWS_PER_W
    lanes = lax.iota(jnp.int32, 16)
    ones = jnp.full((16,), 1, jnp.int32)
    zeros = jnp.full((16,), 0, jnp.int32)

    pltpu.sync_copy(pos_hbm, ptab_v)

    def row_body(i, acc):
        row = row0 + i
        pltpu.sync_copy(ids_hbm.at[pl.ds(row * _L, _L)], idx_v.at[pl.ds(0, _L)])

        carry = zeros
        copies = []
        for j in range(_NCH):
            v = idx_v[pl.ds(j * 16, 16)]
            if j == _NCH - 1:
                v = jnp.where(lanes < _TAIL, v, zeros)
            m = jnp.where(v != 0, ones, zeros)
            pidx_v[pl.ds(j * 16, 16)] = (plsc.cumsum(m) + carry) * m
            carry = carry + lax.broadcast_in_dim(jnp.sum(m), (16,), ())
            copies.append(
                pltpu.async_copy(char_hbm.at[v], cbuf.at[pl.ds(j * 16, 16)], sem))
        for c in copies:
            c.wait()

        def add_chunk(j, a):
            pv = pidx_v[pl.ds(j * 16, 16)]
            base = j * 16
            for t in range(16):
                p = pv[t]
                r = base + t
                for k in range(_D // 16):
                    sl = pl.ds(k * 16, 16)
                    cbuf[r, sl] = cbuf[r, sl] + ptab_v[p, sl]
            return a

        lax.fori_loop(0, _NCH, add_chunk, 0)
        pltpu.sync_copy(cbuf.at[pl.ds(0, _L)], out_hbm.at[row])
        return acc

    lax.fori_loop(0, _ROWS_PER_W, row_body, 0)


def kernel(input_ids, char_table, pos_table):
    return _embed_kernel(input_ids.reshape(-1), char_table, pos_table)
